# Initial kernel scaffold; baseline (speedup 1.0000x reference)
import jax
import jax.numpy as jnp
from jax.experimental import pallas as pl


def kernel(atomic_numbers, positions, batch, emb, blocks, out_w1, out_b1, out_w2, out_b2):
    raise NotImplementedError("write your pallas kernel here")


# speedup vs baseline: 38.0120x; 38.0120x over previous
import jax
import jax.numpy as jnp
from jax.experimental import pallas as pl

N = 512
H = 64
NR = 60
NB = 4
NM = 32
CUTOFF = 5.0
TI = 16
NCH = N // TI
E = TI * N


def _silu(x):
    return x * (1.0 / (1.0 + jnp.exp(-x)))


def _body(an_ref, pos_ref, batc_ref, emb_ref, *rest):
    f32 = jnp.float32
    wrefs = rest[:9 * NB]
    ow1_ref, ob1_ref, ow2_ref, ob2_ref, y_ref = rest[9 * NB:]

    an = jnp.clip(an_ref[...], 0, 99)
    onehot = (jax.lax.broadcasted_iota(jnp.int32, (100, N), 0) == an).astype(f32)
    xT = jnp.dot(emb_ref[...], onehot, preferred_element_type=f32)

    pos = pos_ref[...]
    pcol = jnp.concatenate([pos] * TI, axis=1)

    lane = jax.lax.broadcasted_iota(jnp.int32, (1, E), 1)
    lanediv = lane // N
    lanemod = lane % N
    Rm = (jax.lax.broadcasted_iota(jnp.int32, (TI, E), 0) == lanediv).astype(f32)

    centers = (jax.lax.broadcasted_iota(jnp.int32, (NR, 1), 0).astype(f32)
               * (CUTOFF / (NR - 1)))
    width = CUTOFF / NR
    inv = 1.0 / (2.0 * width * width)

    ei_row = jax.lax.broadcasted_iota(jnp.int32, (N, TI), 0)
    ei_col = jax.lax.broadcasted_iota(jnp.int32, (N, TI), 1)

    def edge_stats(c):
        ec = (ei_row == c * TI + ei_col).astype(f32)
        posc = jnp.dot(pos, ec, preferred_element_type=f32)
        prow = jnp.dot(posc, Rm, preferred_element_type=f32)
        diff = prow - pcol
        d = jnp.sqrt(diff[0:1] ** 2 + diff[1:2] ** 2 + diff[2:3] ** 2)
        keep = ((c * TI + lanediv) != lanemod) & (d < CUTOFF)
        mask = keep.astype(f32)
        rbf = jnp.exp((d - centers) ** 2 * (-inv))
        return ec, mask, rbf

    for b in range(NB):
        w1x, w1r, b1, w2, b2, u1, ub1, u2, ub2 = wrefs[9 * b:9 * (b + 1)]
        xw1 = jnp.dot(w1x[...], xT, preferred_element_type=f32)

        def chunk(c, carry, xw1=xw1, w1r=w1r, b1=b1):
            hsum, cnt = carry
            ec, mask, rbf = edge_stats(c)
            xc = jnp.dot(xw1, ec, preferred_element_type=f32)
            bcast = jnp.dot(xc, Rm, preferred_element_type=f32)
            pre = jnp.dot(w1r[...], rbf, preferred_element_type=f32) + bcast + b1[...]
            hm = _silu(pre) * mask
            for t in range(TI):
                hsum = hsum + hm[:, t * N:(t + 1) * N]
                cnt = cnt + mask[:, t * N:(t + 1) * N]
            return hsum, cnt

        hsum, cnt = jax.lax.fori_loop(
            0, NCH, chunk, (jnp.zeros((H, N), f32), jnp.zeros((1, N), f32)))

        aggr = jnp.dot(w2[...], hsum, preferred_element_type=f32) + b2[...] * cnt
        u = jnp.concatenate([xT, aggr], axis=0)
        hu = _silu(jnp.dot(u1[...], u, preferred_element_type=f32) + ub1[...])
        xT = xT + jnp.dot(u2[...], hu, preferred_element_type=f32) + ub2[...]

    sel = (batc_ref[...] == jax.lax.broadcasted_iota(jnp.int32, (1, NM), 1)).astype(f32)
    mol = jnp.dot(xT, sel, preferred_element_type=f32)
    cntm = jnp.sum(sel, axis=0, keepdims=True)
    mol = mol / jnp.clip(cntm, 1.0, None)
    ho = _silu(jnp.dot(ow1_ref[...], mol, preferred_element_type=f32) + ob1_ref[...])
    y_ref[...] = jnp.dot(ow2_ref[...], ho, preferred_element_type=f32) + ob2_ref[...]


def kernel(atomic_numbers, positions, batch, emb, blocks, out_w1, out_b1, out_w2, out_b2):
    f32 = jnp.float32
    anT = jnp.asarray(atomic_numbers, jnp.int32).reshape(1, N)
    posT = jnp.asarray(positions, f32).T
    batc = jnp.asarray(batch, jnp.int32).reshape(N, 1)
    embT = jnp.asarray(emb, f32).T
    wflat = []
    for blk in blocks:
        wflat += [
            blk['msg_w1'][:H].T, blk['msg_w1'][H:].T, blk['msg_b1'].reshape(H, 1),
            blk['msg_w2'].T, blk['msg_b2'].reshape(H, 1),
            blk['upd_w1'].T, blk['upd_b1'].reshape(H, 1),
            blk['upd_w2'].T, blk['upd_b2'].reshape(H, 1),
        ]
    yT = pl.pallas_call(
        _body,
        out_shape=jax.ShapeDtypeStruct((1, NM), f32),
    )(anT, posT, batc, embT, *wflat,
      out_w1.T, out_b1.reshape(H // 2, 1), out_w2.T, out_b2.reshape(1, 1))
    return yT.reshape(NM, 1)


# speedup vs baseline: 42.5511x; 1.1194x over previous
import math

import jax
import jax.numpy as jnp
from jax.experimental import pallas as pl
from jax.experimental.pallas import tpu as pltpu

N = 512
H = 64
NR = 60
NB = 4
NM = 32
CUTOFF = 5.0
TI = 32
NCH = N // TI
E = TI * N
KG = NR + TI + 1


def _silu(x):
    r = x * 0.5
    return r + r * jnp.tanh(r)


def _body(an_ref, pos_ref, batc_ref, emb_ref, *rest):
    f32 = jnp.float32
    wrefs = rest[:9 * NB]
    ow1_ref, ob1_ref, ow2_ref, ob2_ref, y_ref, g_ref = rest[9 * NB:]

    an = jnp.clip(an_ref[...], 0, 99)
    onehot = (jax.lax.broadcasted_iota(jnp.int32, (100, N), 0) == an).astype(f32)
    xT = jnp.dot(emb_ref[...], onehot, preferred_element_type=f32)

    pos = pos_ref[...]
    pcol = jnp.concatenate([pos] * TI, axis=1)

    lane = jax.lax.broadcasted_iota(jnp.int32, (1, E), 1)
    lanediv = lane // N
    lanemod = lane % N
    Rm = (jax.lax.broadcasted_iota(jnp.int32, (TI, E), 0) == lanediv).astype(f32)
    g_ref[NR:NR + TI, :] = Rm
    g_ref[NR + TI:KG, :] = jnp.ones((1, E), f32)

    width = CUTOFF / NR
    inv = 1.0 / (2.0 * width * width)
    scale = math.sqrt(inv * math.log2(math.e))
    centers_s = (jax.lax.broadcasted_iota(jnp.int32, (NR, 1), 0).astype(f32)
                 * (CUTOFF / (NR - 1) * scale))

    ei_row = jax.lax.broadcasted_iota(jnp.int32, (N, TI), 0)
    ei_col = jax.lax.broadcasted_iota(jnp.int32, (N, TI), 1)

    def edge_stats(c):
        ec = (ei_row == c * TI + ei_col).astype(f32)
        posc = jnp.dot(pos, ec, preferred_element_type=f32)
        prow = jnp.dot(posc, Rm, preferred_element_type=f32)
        diff = prow - pcol
        d = jnp.sqrt(diff[0:1] ** 2 + diff[1:2] ** 2 + diff[2:3] ** 2)
        keep = ((c * TI + lanediv) != lanemod) & (d < CUTOFF)
        mask = keep.astype(f32)
        y = d * scale - centers_s
        rbf = jnp.exp2(-(y * y))
        return ec, mask, rbf

    for b in range(NB):
        w1x, w1r, b1, w2, b2, u1, ub1, u2, ub2 = wrefs[9 * b:9 * (b + 1)]
        xw1 = jnp.dot(w1x[...], xT, preferred_element_type=f32)
        w1rv, b1v = w1r[...], b1[...]

        def chunk(c, carry, xw1=xw1, w1rv=w1rv, b1v=b1v):
            hsum, cnt = carry
            ec, mask, rbf = edge_stats(c)
            g_ref[0:NR, :] = rbf
            xc = jnp.dot(xw1, ec, preferred_element_type=f32)
            wcat = jnp.concatenate([w1rv, xc, b1v], axis=1)
            pre = jnp.dot(wcat, g_ref[...], preferred_element_type=f32)
            hm = _silu(pre) * mask
            for t in range(TI):
                hsum = hsum + hm[:, t * N:(t + 1) * N]
                cnt = cnt + mask[:, t * N:(t + 1) * N]
            return hsum, cnt

        hsum, cnt = jax.lax.fori_loop(
            0, NCH, chunk, (jnp.zeros((H, N), f32), jnp.zeros((1, N), f32)))

        aggr = jnp.dot(w2[...], hsum, preferred_element_type=f32) + b2[...] * cnt
        u = jnp.concatenate([xT, aggr], axis=0)
        hu = _silu(jnp.dot(u1[...], u, preferred_element_type=f32) + ub1[...])
        xT = xT + jnp.dot(u2[...], hu, preferred_element_type=f32) + ub2[...]

    sel = (batc_ref[...] == jax.lax.broadcasted_iota(jnp.int32, (1, NM), 1)).astype(f32)
    mol = jnp.dot(xT, sel, preferred_element_type=f32)
    cntm = jnp.sum(sel, axis=0, keepdims=True)
    mol = mol / jnp.clip(cntm, 1.0, None)
    ho = _silu(jnp.dot(ow1_ref[...], mol, preferred_element_type=f32) + ob1_ref[...])
    y_ref[...] = jnp.dot(ow2_ref[...], ho, preferred_element_type=f32) + ob2_ref[...]


def kernel(atomic_numbers, positions, batch, emb, blocks, out_w1, out_b1, out_w2, out_b2):
    f32 = jnp.float32
    anT = jnp.asarray(atomic_numbers, jnp.int32).reshape(1, N)
    posT = jnp.asarray(positions, f32).T
    batc = jnp.asarray(batch, jnp.int32).reshape(N, 1)
    embT = jnp.asarray(emb, f32).T
    wflat = []
    for blk in blocks:
        wflat += [
            blk['msg_w1'][:H].T, blk['msg_w1'][H:].T, blk['msg_b1'].reshape(H, 1),
            blk['msg_w2'].T, blk['msg_b2'].reshape(H, 1),
            blk['upd_w1'].T, blk['upd_b1'].reshape(H, 1),
            blk['upd_w2'].T, blk['upd_b2'].reshape(H, 1),
        ]
    yT = pl.pallas_call(
        _body,
        out_shape=jax.ShapeDtypeStruct((1, NM), f32),
        scratch_shapes=[pltpu.VMEM((KG, E), f32)],
    )(anT, posT, batc, embT, *wflat,
      out_w1.T, out_b1.reshape(H // 2, 1), out_w2.T, out_b2.reshape(1, 1))
    return yT.reshape(NM, 1)
